# initial kernel scaffold (unmeasured)
import jax
import jax.numpy as jnp
from jax import lax
from jax.experimental import pallas as pl
from jax.experimental.pallas import tpu as pltpu

T_HALF = 1024
D = 1024
F = 4096
F_TILE = 1024
E_LOC = 8
CAP = 384


def _exchange_kernel(x_shard_bf16, router_shard):

    def body(x_ref, r_ref, xo_ref, ro_ref, sx, rx, sr, rr):
        my_x = lax.axis_index("x")
        my_y = lax.axis_index("y")
        my_z = lax.axis_index("z")
        nbr = (1 - my_x, my_y, my_z)

        barrier_sem = pltpu.get_barrier_semaphore()
        pl.semaphore_signal(barrier_sem, inc=1, device_id=nbr,
                            device_id_type=pl.DeviceIdType.MESH)
        pl.semaphore_wait(barrier_sem, 1)

        xo_ref[pl.ds(my_x, 1)] = x_ref[...][None]
        ro_ref[pl.ds(my_x, 1)] = r_ref[...][None]

        rdma_x = pltpu.make_async_remote_copy(
            src_ref=x_ref, dst_ref=xo_ref.at[my_x],
            send_sem=sx, recv_sem=rx,
            device_id=nbr, device_id_type=pl.DeviceIdType.MESH,
        )
        rdma_r = pltpu.make_async_remote_copy(
            src_ref=r_ref, dst_ref=ro_ref.at[my_x],
            send_sem=sr, recv_sem=rr,
            device_id=nbr, device_id_type=pl.DeviceIdType.MESH,
        )
        rdma_x.start()
        rdma_r.start()
        rdma_x.wait()
        rdma_r.wait()

    return pl.pallas_call(
        body,
        out_shape=[
            jax.ShapeDtypeStruct((2, T_HALF, D), jnp.bfloat16),
            jax.ShapeDtypeStruct((2, D, E_LOC), jnp.float32),
        ],
        in_specs=[
            pl.BlockSpec(memory_space=pltpu.VMEM),
            pl.BlockSpec(memory_space=pltpu.VMEM),
        ],
        out_specs=[
            pl.BlockSpec(memory_space=pltpu.VMEM),
            pl.BlockSpec(memory_space=pltpu.VMEM),
        ],
        scratch_shapes=[
            pltpu.SemaphoreType.DMA,
            pltpu.SemaphoreType.DMA,
            pltpu.SemaphoreType.DMA,
            pltpu.SemaphoreType.DMA,
        ],
        compiler_params=pltpu.CompilerParams(collective_id=0),
    )(x_shard_bf16, router_shard)


def _expert_compute(xg, w1, w2):

    def body(xg_ref, w1_ref, w2_ref, y_ref):
        f = pl.program_id(1)
        xge = xg_ref[0]
        h = jnp.dot(xge, w1_ref[0].astype(jnp.bfloat16),
                    preferred_element_type=jnp.float32)
        h = jnp.maximum(h, 0.0).astype(jnp.bfloat16)
        contrib = jnp.dot(h, w2_ref[0].astype(jnp.bfloat16),
                          preferred_element_type=jnp.float32)

        @pl.when(f == 0)
        def _():
            y_ref[0] = contrib

        @pl.when(f != 0)
        def _():
            y_ref[0] += contrib

    n_f = F // F_TILE
    return pl.pallas_call(
        body,
        grid=(E_LOC, n_f),
        in_specs=[
            pl.BlockSpec((1, CAP, D), lambda e, f: (e, 0, 0)),
            pl.BlockSpec((1, D, F_TILE), lambda e, f: (e, 0, f)),
            pl.BlockSpec((1, F_TILE, D), lambda e, f: (e, f, 0)),
        ],
        out_specs=pl.BlockSpec((1, CAP, D), lambda e, f: (e, 0, 0)),
        out_shape=jax.ShapeDtypeStruct((E_LOC, CAP, D), jnp.float32),
        compiler_params=pltpu.CompilerParams(
            dimension_semantics=("parallel", "arbitrary"),
        ),
    )(xg, w1, w2)


def _combine_kernel(partial_bf16):

    def body(p_ref, out_ref, comm_ref, s_sem, r_sem):
        my_x = lax.axis_index("x")
        my_y = lax.axis_index("y")
        my_z = lax.axis_index("z")
        nbr = (1 - my_x, my_y, my_z)

        barrier_sem = pltpu.get_barrier_semaphore()
        pl.semaphore_signal(barrier_sem, inc=1, device_id=nbr,
                            device_id_type=pl.DeviceIdType.MESH)
        pl.semaphore_wait(barrier_sem, 1)

        rdma = pltpu.make_async_remote_copy(
            src_ref=p_ref.at[1 - my_x], dst_ref=comm_ref,
            send_sem=s_sem, recv_sem=r_sem,
            device_id=nbr, device_id_type=pl.DeviceIdType.MESH,
        )
        rdma.start()
        rdma.wait()

        mine = p_ref[pl.ds(my_x, 1)][0]
        out_ref[...] = mine.astype(jnp.float32) + comm_ref[...].astype(jnp.float32)

    return pl.pallas_call(
        body,
        out_shape=jax.ShapeDtypeStruct((T_HALF, D), jnp.float32),
        in_specs=[pl.BlockSpec(memory_space=pltpu.VMEM)],
        out_specs=pl.BlockSpec(memory_space=pltpu.VMEM),
        scratch_shapes=[
            pltpu.VMEM((T_HALF, D), jnp.bfloat16),
            pltpu.SemaphoreType.DMA,
            pltpu.SemaphoreType.DMA,
        ],
        compiler_params=pltpu.CompilerParams(collective_id=1),
    )(partial_bf16)


def kernel(x, router, W1, W2):
    my_x = lax.axis_index("x")

    x_all, r_all = _exchange_kernel(x.astype(jnp.bfloat16), router)
    x_all = x_all.reshape(2 * T_HALF, D)
    router_full = jnp.concatenate([r_all[0], r_all[1]], axis=1)

    gates = jnp.dot(x_all, router_full.astype(jnp.bfloat16),
                    preferred_element_type=jnp.float32)
    top1 = jnp.argmax(gates, axis=1)
    g1 = jnp.max(gates, axis=1)
    masked = gates - 1e30 * jax.nn.one_hot(top1, 16, dtype=gates.dtype)
    top2 = jnp.argmax(masked, axis=1)
    g2 = jnp.max(masked, axis=1)
    w1g = 1.0 / (1.0 + jnp.exp(g2 - g1))
    w2g = 1.0 - w1g

    eids = my_x * E_LOC + jnp.arange(E_LOC)
    ind = (top1[:, None] == eids[None, :]) | (top2[:, None] == eids[None, :])
    order = jnp.argsort(jnp.where(ind, 0, 1), axis=0, stable=True)
    idx = order[:CAP].T
    cnt = jnp.sum(ind, axis=0)
    valid = jnp.arange(CAP)[None, :] < cnt[:, None]
    w_tok = jnp.where(top1[:, None] == eids[None, :], w1g[:, None], w2g[:, None])
    w_sel = jnp.take_along_axis(w_tok, idx.T, axis=0).T
    w_sel = jnp.where(valid, w_sel, 0.0)

    xg = x_all[idx]

    y = _expert_compute(xg, W1, W2)
    y = y * w_sel[:, :, None]

    partial = jnp.zeros((2 * T_HALF, D), jnp.float32)
    partial = partial.at[idx.reshape(-1)].add(y.reshape(-1, D))
    partial = partial.astype(jnp.bfloat16).reshape(2, T_HALF, D)

    return _combine_kernel(partial)


# baseline (device time: 231966 ns/iter reference)
import jax
import jax.numpy as jnp
from jax import lax
from jax.experimental import pallas as pl
from jax.experimental.pallas import tpu as pltpu

T_HALF = 1024
D = 1024
F = 4096
F_TILE = 1024
E_LOC = 8
CAP = 384


def _exchange_kernel(x_shard, router_shard):

    def body(x_ref, r_ref, xo_ref, go_ref, xsend, rfull,
             sx, rx, sr, rr, sg, rg):
        my_x = lax.axis_index("x")
        my_y = lax.axis_index("y")
        my_z = lax.axis_index("z")
        nbr = (1 - my_x, my_y, my_z)

        barrier_sem = pltpu.get_barrier_semaphore()
        pl.semaphore_signal(barrier_sem, inc=1, device_id=nbr,
                            device_id_type=pl.DeviceIdType.MESH)
        pl.semaphore_wait(barrier_sem, 1)

        xsend[...] = x_ref[...].astype(jnp.bfloat16)
        rdma_x = pltpu.make_async_remote_copy(
            src_ref=xsend, dst_ref=xo_ref.at[my_x],
            send_sem=sx, recv_sem=rx,
            device_id=nbr, device_id_type=pl.DeviceIdType.MESH,
        )
        rdma_x.start()
        xo_ref[pl.ds(my_x, 1)] = xsend[...][None]

        rfull[pl.ds(my_x, 1)] = r_ref[...][None]
        rdma_r = pltpu.make_async_remote_copy(
            src_ref=r_ref, dst_ref=rfull.at[my_x],
            send_sem=sr, recv_sem=rr,
            device_id=nbr, device_id_type=pl.DeviceIdType.MESH,
        )
        rdma_r.start()
        rdma_r.wait()

        router_full = jnp.concatenate([rfull[0], rfull[1]], axis=1)
        g_mine = jnp.dot(x_ref[...], router_full,
                         preferred_element_type=jnp.float32,
                         precision=lax.Precision.HIGHEST)
        go_ref[pl.ds(my_x, 1)] = g_mine[None]
        rdma_g = pltpu.make_async_remote_copy(
            src_ref=go_ref.at[my_x], dst_ref=go_ref.at[my_x],
            send_sem=sg, recv_sem=rg,
            device_id=nbr, device_id_type=pl.DeviceIdType.MESH,
        )
        rdma_g.start()
        rdma_g.wait()
        rdma_x.wait()

    return pl.pallas_call(
        body,
        out_shape=[
            jax.ShapeDtypeStruct((2, T_HALF, D), jnp.bfloat16),
            jax.ShapeDtypeStruct((2, T_HALF, 2 * E_LOC), jnp.float32),
        ],
        in_specs=[
            pl.BlockSpec(memory_space=pltpu.VMEM),
            pl.BlockSpec(memory_space=pltpu.VMEM),
        ],
        out_specs=[
            pl.BlockSpec(memory_space=pltpu.VMEM),
            pl.BlockSpec(memory_space=pltpu.VMEM),
        ],
        scratch_shapes=[
            pltpu.VMEM((T_HALF, D), jnp.bfloat16),
            pltpu.VMEM((2, D, E_LOC), jnp.float32),
            pltpu.SemaphoreType.DMA,
            pltpu.SemaphoreType.DMA,
            pltpu.SemaphoreType.DMA,
            pltpu.SemaphoreType.DMA,
            pltpu.SemaphoreType.DMA,
            pltpu.SemaphoreType.DMA,
        ],
        compiler_params=pltpu.CompilerParams(collective_id=0),
    )(x_shard, router_shard)


def _expert_compute(xg, w1, w2):

    def body(xg_ref, w1_ref, w2_ref, y_ref):
        f = pl.program_id(1)
        xge = xg_ref[0]
        h = jnp.dot(xge, w1_ref[0].astype(jnp.bfloat16),
                    preferred_element_type=jnp.float32)
        h = jnp.maximum(h, 0.0).astype(jnp.bfloat16)
        contrib = jnp.dot(h, w2_ref[0].astype(jnp.bfloat16),
                          preferred_element_type=jnp.float32)

        @pl.when(f == 0)
        def _():
            y_ref[0] = contrib

        @pl.when(f != 0)
        def _():
            y_ref[0] += contrib

    n_f = F // F_TILE
    return pl.pallas_call(
        body,
        grid=(E_LOC, n_f),
        in_specs=[
            pl.BlockSpec((1, CAP, D), lambda e, f: (e, 0, 0)),
            pl.BlockSpec((1, D, F_TILE), lambda e, f: (e, 0, f)),
            pl.BlockSpec((1, F_TILE, D), lambda e, f: (e, f, 0)),
        ],
        out_specs=pl.BlockSpec((1, CAP, D), lambda e, f: (e, 0, 0)),
        out_shape=jax.ShapeDtypeStruct((E_LOC, CAP, D), jnp.float32),
        compiler_params=pltpu.CompilerParams(
            dimension_semantics=("parallel", "arbitrary"),
        ),
    )(xg, w1, w2)


def _combine_kernel(partial_bf16):

    def body(p_ref, out_ref, comm_ref, s_sem, r_sem):
        my_x = lax.axis_index("x")
        my_y = lax.axis_index("y")
        my_z = lax.axis_index("z")
        nbr = (1 - my_x, my_y, my_z)

        barrier_sem = pltpu.get_barrier_semaphore()
        pl.semaphore_signal(barrier_sem, inc=1, device_id=nbr,
                            device_id_type=pl.DeviceIdType.MESH)
        pl.semaphore_wait(barrier_sem, 1)

        rdma = pltpu.make_async_remote_copy(
            src_ref=p_ref.at[1 - my_x], dst_ref=comm_ref,
            send_sem=s_sem, recv_sem=r_sem,
            device_id=nbr, device_id_type=pl.DeviceIdType.MESH,
        )
        rdma.start()
        rdma.wait()

        mine = p_ref[pl.ds(my_x, 1)][0]
        out_ref[...] = mine.astype(jnp.float32) + comm_ref[...].astype(jnp.float32)

    return pl.pallas_call(
        body,
        out_shape=jax.ShapeDtypeStruct((T_HALF, D), jnp.float32),
        in_specs=[pl.BlockSpec(memory_space=pltpu.VMEM)],
        out_specs=pl.BlockSpec(memory_space=pltpu.VMEM),
        scratch_shapes=[
            pltpu.VMEM((T_HALF, D), jnp.bfloat16),
            pltpu.SemaphoreType.DMA,
            pltpu.SemaphoreType.DMA,
        ],
        compiler_params=pltpu.CompilerParams(collective_id=1),
    )(partial_bf16)


def kernel(x, router, W1, W2):
    my_x = lax.axis_index("x")

    x_all, g_all = _exchange_kernel(x, router)
    x_all = x_all.reshape(2 * T_HALF, D)
    gates = g_all.reshape(2 * T_HALF, 2 * E_LOC)
    top1 = jnp.argmax(gates, axis=1)
    g1 = jnp.max(gates, axis=1)
    masked = gates - 1e30 * jax.nn.one_hot(top1, 16, dtype=gates.dtype)
    top2 = jnp.argmax(masked, axis=1)
    g2 = jnp.max(masked, axis=1)
    w1g = 1.0 / (1.0 + jnp.exp(g2 - g1))
    w2g = 1.0 - w1g

    eids = my_x * E_LOC + jnp.arange(E_LOC)
    ind = (top1[:, None] == eids[None, :]) | (top2[:, None] == eids[None, :])
    order = jnp.argsort(jnp.where(ind, 0, 1), axis=0, stable=True)
    idx = order[:CAP].T
    cnt = jnp.sum(ind, axis=0)
    valid = jnp.arange(CAP)[None, :] < cnt[:, None]
    w_tok = jnp.where(top1[:, None] == eids[None, :], w1g[:, None], w2g[:, None])
    w_sel = jnp.take_along_axis(w_tok, idx.T, axis=0).T
    w_sel = jnp.where(valid, w_sel, 0.0)

    xg = x_all[idx]

    y = _expert_compute(xg, W1, W2)
    y = y * w_sel[:, :, None]

    partial = jnp.zeros((2 * T_HALF, D), jnp.float32)
    partial = partial.at[idx.reshape(-1)].add(y.reshape(-1, D))
    partial = partial.astype(jnp.bfloat16).reshape(2, T_HALF, D)

    return _combine_kernel(partial)


# device time: 229644 ns/iter; 1.0101x vs baseline; 1.0101x over previous
import jax
import jax.numpy as jnp
from jax import lax
from jax.experimental import pallas as pl
from jax.experimental.pallas import tpu as pltpu

T_HALF = 1024
T = 2 * T_HALF
D = 1024
F = 4096
F_TILE = 512
N_F = F // F_TILE
E_LOC = 8
N_E = 2 * E_LOC
CAP = 320


def _exchange_kernel(x_shard, router_shard):

    def body(x_ref, r_ref, xo_ref, go_ref, xsend, rfull,
             sx, rx, sr, rr, sg, rg):
        my_x = lax.axis_index("x")
        my_y = lax.axis_index("y")
        my_z = lax.axis_index("z")
        nbr = (1 - my_x, my_y, my_z)

        barrier_sem = pltpu.get_barrier_semaphore()
        pl.semaphore_signal(barrier_sem, inc=1, device_id=nbr,
                            device_id_type=pl.DeviceIdType.MESH)
        pl.semaphore_wait(barrier_sem, 1)

        xsend[...] = x_ref[...].astype(jnp.bfloat16)
        rdma_x = pltpu.make_async_remote_copy(
            src_ref=xsend, dst_ref=xo_ref.at[pl.ds(my_x * T_HALF, T_HALF)],
            send_sem=sx, recv_sem=rx,
            device_id=nbr, device_id_type=pl.DeviceIdType.MESH,
        )
        rdma_x.start()
        xo_ref[pl.ds(my_x * T_HALF, T_HALF), :] = xsend[...]

        rfull[pl.ds(my_x, 1)] = r_ref[...][None]
        rdma_r = pltpu.make_async_remote_copy(
            src_ref=r_ref, dst_ref=rfull.at[my_x],
            send_sem=sr, recv_sem=rr,
            device_id=nbr, device_id_type=pl.DeviceIdType.MESH,
        )
        rdma_r.start()
        rdma_r.wait()

        router_full = jnp.concatenate([rfull[0], rfull[1]], axis=1)
        g_mine = jnp.dot(x_ref[...], router_full,
                         preferred_element_type=jnp.float32,
                         precision=lax.Precision.HIGHEST)
        go_ref[pl.ds(my_x * T_HALF, T_HALF), :] = g_mine
        rdma_g = pltpu.make_async_remote_copy(
            src_ref=go_ref.at[pl.ds(my_x * T_HALF, T_HALF)],
            dst_ref=go_ref.at[pl.ds(my_x * T_HALF, T_HALF)],
            send_sem=sg, recv_sem=rg,
            device_id=nbr, device_id_type=pl.DeviceIdType.MESH,
        )
        rdma_g.start()
        rdma_g.wait()
        rdma_x.wait()

    return pl.pallas_call(
        body,
        out_shape=[
            jax.ShapeDtypeStruct((T, D), jnp.bfloat16),
            jax.ShapeDtypeStruct((T, N_E), jnp.float32),
        ],
        in_specs=[
            pl.BlockSpec(memory_space=pltpu.VMEM),
            pl.BlockSpec(memory_space=pltpu.VMEM),
        ],
        out_specs=[
            pl.BlockSpec(memory_space=pltpu.VMEM),
            pl.BlockSpec(memory_space=pltpu.VMEM),
        ],
        scratch_shapes=[
            pltpu.VMEM((T_HALF, D), jnp.bfloat16),
            pltpu.VMEM((2, D, E_LOC), jnp.float32),
            pltpu.SemaphoreType.DMA,
            pltpu.SemaphoreType.DMA,
            pltpu.SemaphoreType.DMA,
            pltpu.SemaphoreType.DMA,
            pltpu.SemaphoreType.DMA,
            pltpu.SemaphoreType.DMA,
        ],
        compiler_params=pltpu.CompilerParams(collective_id=0),
    )(x_shard, router_shard)


def _moe_kernel(x_all, gates, W1, W2):

    def body(x_ref, g_ref, w1_ref, w2_ref, out_ref,
             d_mat, w_tok_s, xg, y_acc, partial, comm, s_sem, r_sem):
        e = pl.program_id(0)
        f = pl.program_id(1)
        my_x = lax.axis_index("x")
        my_y = lax.axis_index("y")
        my_z = lax.axis_index("z")
        nbr = (1 - my_x, my_y, my_z)

        @pl.when(f == 0)
        def _route():
            g = g_ref[...]
            ids = lax.broadcasted_iota(jnp.int32, (T, N_E), 1)
            m1 = jnp.max(g, axis=1, keepdims=True)
            top1 = jnp.argmax(g, axis=1).reshape(T, 1)
            masked = jnp.where(ids == top1, -jnp.inf, g)
            m2 = jnp.max(masked, axis=1, keepdims=True)
            top2 = jnp.argmax(masked, axis=1).reshape(T, 1)
            w1g = 1.0 / (1.0 + jnp.exp(m2 - m1))
            w2g = 1.0 - w1g

            e_glob = my_x * E_LOC + e
            sel1 = top1 == e_glob
            sel2 = top2 == e_glob
            ind = sel1 | sel2
            w_tok = jnp.where(sel1, w1g, 0.0) + jnp.where(sel2, w2g, 0.0)
            w_tok_s[...] = w_tok

            pos = ind.astype(jnp.int32)
            k = 1
            while k < T:
                shifted = jnp.concatenate(
                    [jnp.zeros((k, 1), jnp.int32), pos[:-k]], axis=0)
                pos = pos + shifted
                k *= 2
            slot_of_tok = jnp.transpose(pos - 1)
            ind_t = jnp.transpose(ind)
            cap_ids = lax.broadcasted_iota(jnp.int32, (CAP, T), 0)
            d_bool = (cap_ids == slot_of_tok) & ind_t
            d_mat[...] = d_bool.astype(jnp.bfloat16)

            xg[...] = jnp.dot(d_mat[...], x_ref[...],
                              preferred_element_type=jnp.float32
                              ).astype(jnp.bfloat16)

        h = jnp.dot(xg[...], w1_ref[0].astype(jnp.bfloat16),
                    preferred_element_type=jnp.float32)
        h = jnp.maximum(h, 0.0).astype(jnp.bfloat16)
        contrib = jnp.dot(h, w2_ref[0].astype(jnp.bfloat16),
                          preferred_element_type=jnp.float32)

        @pl.when(f == 0)
        def _():
            y_acc[...] = contrib

        @pl.when(f != 0)
        def _():
            y_acc[...] += contrib

        @pl.when(f == N_F - 1)
        def _scatter():
            yb = y_acc[...].astype(jnp.bfloat16)
            s = lax.dot_general(d_mat[...], yb,
                                dimension_numbers=(((0,), (0,)), ((), ())),
                                preferred_element_type=jnp.float32)
            s = s * w_tok_s[...]

            @pl.when(e == 0)
            def _():
                partial[...] = s.astype(jnp.bfloat16)

            @pl.when(e != 0)
            def _():
                partial[...] = (partial[...].astype(jnp.float32) + s
                                ).astype(jnp.bfloat16)

        @pl.when((e == E_LOC - 1) & (f == N_F - 1))
        def _combine():
            barrier_sem = pltpu.get_barrier_semaphore()
            pl.semaphore_signal(barrier_sem, inc=1, device_id=nbr,
                                device_id_type=pl.DeviceIdType.MESH)
            pl.semaphore_wait(barrier_sem, 1)

            rdma = pltpu.make_async_remote_copy(
                src_ref=partial.at[pl.ds((1 - my_x) * T_HALF, T_HALF)],
                dst_ref=comm,
                send_sem=s_sem, recv_sem=r_sem,
                device_id=nbr, device_id_type=pl.DeviceIdType.MESH,
            )
            rdma.start()
            rdma.wait()

            mine = partial[pl.ds(my_x * T_HALF, T_HALF), :]
            out_ref[...] = mine.astype(jnp.float32) + comm[...].astype(
                jnp.float32)

    return pl.pallas_call(
        body,
        grid=(E_LOC, N_F),
        in_specs=[
            pl.BlockSpec((T, D), lambda e, f: (0, 0)),
            pl.BlockSpec((T, N_E), lambda e, f: (0, 0)),
            pl.BlockSpec((1, D, F_TILE), lambda e, f: (e, 0, f)),
            pl.BlockSpec((1, F_TILE, D), lambda e, f: (e, f, 0)),
        ],
        out_specs=pl.BlockSpec((T_HALF, D), lambda e, f: (0, 0)),
        out_shape=jax.ShapeDtypeStruct((T_HALF, D), jnp.float32),
        scratch_shapes=[
            pltpu.VMEM((CAP, T), jnp.bfloat16),
            pltpu.VMEM((T, 1), jnp.float32),
            pltpu.VMEM((CAP, D), jnp.bfloat16),
            pltpu.VMEM((CAP, D), jnp.float32),
            pltpu.VMEM((T, D), jnp.bfloat16),
            pltpu.VMEM((T_HALF, D), jnp.bfloat16),
            pltpu.SemaphoreType.DMA,
            pltpu.SemaphoreType.DMA,
        ],
        compiler_params=pltpu.CompilerParams(
            collective_id=1,
            dimension_semantics=("arbitrary", "arbitrary"),
        ),
    )(x_all, gates, W1, W2)


def kernel(x, router, W1, W2):
    x_all, gates = _exchange_kernel(x, router)
    return _moe_kernel(x_all, gates, W1, W2)


# device time: 191806 ns/iter; 1.2094x vs baseline; 1.1973x over previous
import jax
import jax.numpy as jnp
from jax import lax
from jax.experimental import pallas as pl
from jax.experimental.pallas import tpu as pltpu

T_HALF = 1024
T = 2 * T_HALF
D = 1024
F = 4096
F_TILE = 1024
N_F = F // F_TILE
E_LOC = 8
N_E = 2 * E_LOC
CAP = 320


def _exchange_kernel(x_shard, router_shard):

    def body(x_ref, r_ref, xo_ref, go_ref, xsend, rfull,
             sx, rx, sr, rr, sg, rg):
        my_x = lax.axis_index("x")
        my_y = lax.axis_index("y")
        my_z = lax.axis_index("z")
        nbr = (1 - my_x, my_y, my_z)

        barrier_sem = pltpu.get_barrier_semaphore()
        pl.semaphore_signal(barrier_sem, inc=1, device_id=nbr,
                            device_id_type=pl.DeviceIdType.MESH)
        pl.semaphore_wait(barrier_sem, 1)

        xsend[...] = x_ref[...].astype(jnp.bfloat16)
        rdma_x = pltpu.make_async_remote_copy(
            src_ref=xsend, dst_ref=xo_ref.at[pl.ds(my_x * T_HALF, T_HALF)],
            send_sem=sx, recv_sem=rx,
            device_id=nbr, device_id_type=pl.DeviceIdType.MESH,
        )
        rdma_x.start()
        xo_ref[pl.ds(my_x * T_HALF, T_HALF), :] = xsend[...]

        rfull[pl.ds(my_x, 1)] = r_ref[...][None]
        rdma_r = pltpu.make_async_remote_copy(
            src_ref=r_ref, dst_ref=rfull.at[my_x],
            send_sem=sr, recv_sem=rr,
            device_id=nbr, device_id_type=pl.DeviceIdType.MESH,
        )
        rdma_r.start()
        rdma_r.wait()

        router_full = jnp.concatenate([rfull[0], rfull[1]], axis=1)
        g_mine = jnp.dot(x_ref[...], router_full,
                         preferred_element_type=jnp.float32,
                         precision=lax.Precision.HIGHEST)
        go_ref[pl.ds(my_x * T_HALF, T_HALF), :] = g_mine
        rdma_g = pltpu.make_async_remote_copy(
            src_ref=go_ref.at[pl.ds(my_x * T_HALF, T_HALF)],
            dst_ref=go_ref.at[pl.ds(my_x * T_HALF, T_HALF)],
            send_sem=sg, recv_sem=rg,
            device_id=nbr, device_id_type=pl.DeviceIdType.MESH,
        )
        rdma_g.start()
        rdma_g.wait()
        rdma_x.wait()

    return pl.pallas_call(
        body,
        out_shape=[
            jax.ShapeDtypeStruct((T, D), jnp.bfloat16),
            jax.ShapeDtypeStruct((T, N_E), jnp.float32),
        ],
        in_specs=[
            pl.BlockSpec(memory_space=pltpu.VMEM),
            pl.BlockSpec(memory_space=pltpu.VMEM),
        ],
        out_specs=[
            pl.BlockSpec(memory_space=pltpu.VMEM),
            pl.BlockSpec(memory_space=pltpu.VMEM),
        ],
        scratch_shapes=[
            pltpu.VMEM((T_HALF, D), jnp.bfloat16),
            pltpu.VMEM((2, D, E_LOC), jnp.float32),
            pltpu.SemaphoreType.DMA,
            pltpu.SemaphoreType.DMA,
            pltpu.SemaphoreType.DMA,
            pltpu.SemaphoreType.DMA,
            pltpu.SemaphoreType.DMA,
            pltpu.SemaphoreType.DMA,
        ],
        compiler_params=pltpu.CompilerParams(collective_id=0),
    )(x_shard, router_shard)


def _moe_kernel(x_all, gates, W1, W2):

    def body(x_ref, g_ref, w1_ref, w2_ref, partial_ref,
             d_mat, slot_t, w_t, xg, y_acc):
        e = pl.program_id(0)
        f = pl.program_id(1)
        my_x = lax.axis_index("x")

        @pl.when((e == 0) & (f == 0))
        def _route_once():
            g_t = jnp.transpose(g_ref[...])
            ids = lax.broadcasted_iota(jnp.int32, (N_E, T), 0)
            m1 = jnp.max(g_t, axis=0, keepdims=True)
            top1 = jnp.argmax(g_t, axis=0).reshape(1, T)
            masked = jnp.where(ids == top1, -jnp.inf, g_t)
            m2 = jnp.max(masked, axis=0, keepdims=True)
            top2 = jnp.argmax(masked, axis=0).reshape(1, T)
            w1g = 1.0 / (1.0 + jnp.exp(m2 - m1))
            w2g = 1.0 - w1g

            eids = lax.broadcasted_iota(jnp.int32, (E_LOC, T), 0) \
                + my_x * E_LOC
            sel1 = top1 == eids
            sel2 = top2 == eids
            ind = sel1 | sel2
            w_t[...] = jnp.where(sel1, w1g, 0.0) + jnp.where(sel2, w2g, 0.0)

            pos = ind.astype(jnp.int32)
            k = 1
            while k < T:
                shifted = jnp.concatenate(
                    [jnp.zeros((E_LOC, k), jnp.int32), pos[:, :-k]], axis=1)
                pos = pos + shifted
                k *= 2
            slot_t[...] = jnp.where(ind, pos - 1, -1)

        @pl.when(f == 0)
        def _gather():
            cap_ids = lax.broadcasted_iota(jnp.int32, (CAP, T), 0)
            d_bool = cap_ids == slot_t[pl.ds(e, 1)]
            d_mat[...] = d_bool.astype(jnp.bfloat16)
            xg[...] = jnp.dot(d_mat[...], x_ref[...],
                              preferred_element_type=jnp.float32
                              ).astype(jnp.bfloat16)

        h = jnp.dot(xg[...], w1_ref[0].astype(jnp.bfloat16),
                    preferred_element_type=jnp.float32)
        h = jnp.maximum(h, 0.0).astype(jnp.bfloat16)
        contrib = jnp.dot(h, w2_ref[0].astype(jnp.bfloat16),
                          preferred_element_type=jnp.float32)

        @pl.when(f == 0)
        def _():
            y_acc[...] = contrib

        @pl.when(f != 0)
        def _():
            y_acc[...] += contrib

        @pl.when(f == N_F - 1)
        def _scatter():
            w_row = w_t[pl.ds(e, 1)].astype(jnp.bfloat16)
            d_mat[...] = d_mat[...] * w_row
            yb = y_acc[...].astype(jnp.bfloat16)
            s = lax.dot_general(d_mat[...], yb,
                                dimension_numbers=(((0,), (0,)), ((), ())),
                                preferred_element_type=jnp.float32)

            @pl.when(e == 0)
            def _():
                partial_ref[...] = s.astype(jnp.bfloat16)

            @pl.when(e != 0)
            def _():
                partial_ref[...] = (
                    partial_ref[...].astype(jnp.float32) + s
                ).astype(jnp.bfloat16)

    return pl.pallas_call(
        body,
        grid=(E_LOC, N_F),
        in_specs=[
            pl.BlockSpec((T, D), lambda e, f: (0, 0)),
            pl.BlockSpec((T, N_E), lambda e, f: (0, 0)),
            pl.BlockSpec((1, D, F_TILE), lambda e, f: (e, 0, f)),
            pl.BlockSpec((1, F_TILE, D), lambda e, f: (e, f, 0)),
        ],
        out_specs=pl.BlockSpec((T, D), lambda e, f: (0, 0)),
        out_shape=jax.ShapeDtypeStruct((T, D), jnp.bfloat16),
        scratch_shapes=[
            pltpu.VMEM((CAP, T), jnp.bfloat16),
            pltpu.VMEM((E_LOC, T), jnp.int32),
            pltpu.VMEM((E_LOC, T), jnp.float32),
            pltpu.VMEM((CAP, D), jnp.bfloat16),
            pltpu.VMEM((CAP, D), jnp.float32),
        ],
        compiler_params=pltpu.CompilerParams(
            dimension_semantics=("arbitrary", "arbitrary"),
        ),
    )(x_all, gates, W1, W2)


def _combine_kernel(partial):

    def body(p_ref, out_ref, comm, s_sem, r_sem):
        my_x = lax.axis_index("x")
        my_y = lax.axis_index("y")
        my_z = lax.axis_index("z")
        nbr = (1 - my_x, my_y, my_z)

        barrier_sem = pltpu.get_barrier_semaphore()
        pl.semaphore_signal(barrier_sem, inc=1, device_id=nbr,
                            device_id_type=pl.DeviceIdType.MESH)
        pl.semaphore_wait(barrier_sem, 1)

        rdma = pltpu.make_async_remote_copy(
            src_ref=p_ref.at[pl.ds((1 - my_x) * T_HALF, T_HALF)],
            dst_ref=comm,
            send_sem=s_sem, recv_sem=r_sem,
            device_id=nbr, device_id_type=pl.DeviceIdType.MESH,
        )
        rdma.start()
        rdma.wait()

        mine = p_ref[pl.ds(my_x * T_HALF, T_HALF), :]
        out_ref[...] = mine.astype(jnp.float32) + comm[...].astype(jnp.float32)

    return pl.pallas_call(
        body,
        out_shape=jax.ShapeDtypeStruct((T_HALF, D), jnp.float32),
        in_specs=[pl.BlockSpec(memory_space=pltpu.VMEM)],
        out_specs=pl.BlockSpec(memory_space=pltpu.VMEM),
        scratch_shapes=[
            pltpu.VMEM((T_HALF, D), jnp.bfloat16),
            pltpu.SemaphoreType.DMA,
            pltpu.SemaphoreType.DMA,
        ],
        compiler_params=pltpu.CompilerParams(collective_id=1),
    )(partial)


def kernel(x, router, W1, W2):
    x_all, gates = _exchange_kernel(x, router)
    partial = _moe_kernel(x_all, gates, W1, W2)
    return _combine_kernel(partial)


# device time: 176656 ns/iter; 1.3131x vs baseline; 1.0858x over previous
import jax
import jax.numpy as jnp
from jax import lax
from jax.experimental import pallas as pl
from jax.experimental.pallas import tpu as pltpu

T_HALF = 1024
T = 2 * T_HALF
D = 1024
F = 4096
F_TILE = 1024
N_F = F // F_TILE
E_LOC = 8
N_E = 2 * E_LOC
CAP = 320


def _exchange_kernel(x_shard, router_shard):

    def body(x_ref, r_ref, xo_ref, go_ref, xsend, rfull,
             sx, rx, sr, rr, sg, rg):
        my_x = lax.axis_index("x")
        my_y = lax.axis_index("y")
        my_z = lax.axis_index("z")
        nbr = (1 - my_x, my_y, my_z)

        barrier_sem = pltpu.get_barrier_semaphore()
        pl.semaphore_signal(barrier_sem, inc=1, device_id=nbr,
                            device_id_type=pl.DeviceIdType.MESH)
        pl.semaphore_wait(barrier_sem, 1)

        xsend[...] = x_ref[...].astype(jnp.bfloat16)
        rdma_x = pltpu.make_async_remote_copy(
            src_ref=xsend, dst_ref=xo_ref.at[pl.ds(my_x * T_HALF, T_HALF)],
            send_sem=sx, recv_sem=rx,
            device_id=nbr, device_id_type=pl.DeviceIdType.MESH,
        )
        rdma_x.start()
        xo_ref[pl.ds(my_x * T_HALF, T_HALF), :] = xsend[...]

        rfull[pl.ds(my_x, 1)] = r_ref[...][None]
        rdma_r = pltpu.make_async_remote_copy(
            src_ref=r_ref, dst_ref=rfull.at[my_x],
            send_sem=sr, recv_sem=rr,
            device_id=nbr, device_id_type=pl.DeviceIdType.MESH,
        )
        rdma_r.start()
        rdma_r.wait()

        router_full = jnp.concatenate([rfull[0], rfull[1]], axis=1)
        g_mine = jnp.dot(x_ref[...], router_full,
                         preferred_element_type=jnp.float32,
                         precision=lax.Precision.HIGHEST)
        go_ref[pl.ds(my_x * T_HALF, T_HALF), :] = g_mine
        rdma_g = pltpu.make_async_remote_copy(
            src_ref=go_ref.at[pl.ds(my_x * T_HALF, T_HALF)],
            dst_ref=go_ref.at[pl.ds(my_x * T_HALF, T_HALF)],
            send_sem=sg, recv_sem=rg,
            device_id=nbr, device_id_type=pl.DeviceIdType.MESH,
        )
        rdma_g.start()
        rdma_g.wait()
        rdma_x.wait()

    return pl.pallas_call(
        body,
        out_shape=[
            jax.ShapeDtypeStruct((T, D), jnp.bfloat16),
            jax.ShapeDtypeStruct((T, N_E), jnp.float32),
        ],
        in_specs=[
            pl.BlockSpec(memory_space=pltpu.VMEM),
            pl.BlockSpec(memory_space=pltpu.VMEM),
        ],
        out_specs=[
            pl.BlockSpec(memory_space=pltpu.VMEM),
            pl.BlockSpec(memory_space=pltpu.VMEM),
        ],
        scratch_shapes=[
            pltpu.VMEM((T_HALF, D), jnp.bfloat16),
            pltpu.VMEM((2, D, E_LOC), jnp.float32),
            pltpu.SemaphoreType.DMA,
            pltpu.SemaphoreType.DMA,
            pltpu.SemaphoreType.DMA,
            pltpu.SemaphoreType.DMA,
            pltpu.SemaphoreType.DMA,
            pltpu.SemaphoreType.DMA,
        ],
        compiler_params=pltpu.CompilerParams(collective_id=0),
    )(x_shard, router_shard)


def _moe_kernel(x_all, gates, W1, W2, q):

    def body(q_ref, x_ref, g_ref, w1_ref, w2_ref, partial_ref,
             d_mat, slot_t, w_t, xg):
        e = pl.program_id(0)
        my_x = lax.axis_index("x")

        @pl.when(e == 0)
        def _route_once():
            g_t = jnp.transpose(g_ref[...])
            ids = lax.broadcasted_iota(jnp.int32, (N_E, T), 0)
            m1 = jnp.max(g_t, axis=0, keepdims=True)
            top1 = jnp.argmax(g_t, axis=0).reshape(1, T)
            masked = jnp.where(ids == top1, -jnp.inf, g_t)
            m2 = jnp.max(masked, axis=0, keepdims=True)
            top2 = jnp.argmax(masked, axis=0).reshape(1, T)
            w1g = 1.0 / (1.0 + jnp.exp(m2 - m1))
            w2g = 1.0 - w1g

            eids = lax.broadcasted_iota(jnp.int32, (E_LOC, T), 0) \
                + my_x * E_LOC
            sel1 = top1 == eids
            sel2 = top2 == eids
            ind = sel1 | sel2
            w_t[...] = jnp.where(sel1, w1g, 0.0) + jnp.where(sel2, w2g, 0.0)

            pos = ind.astype(jnp.int32)
            k = 1
            while k < T:
                shifted = jnp.concatenate(
                    [jnp.zeros((E_LOC, k), jnp.int32), pos[:, :-k]], axis=1)
                pos = pos + shifted
                k *= 2
            slot_t[...] = jnp.where(ind, pos - 1, -1)

        cap_ids = lax.broadcasted_iota(jnp.int32, (CAP, T), 0)
        d_bool = cap_ids == slot_t[pl.ds(e, 1)]
        d_mat[...] = d_bool.astype(jnp.bfloat16)
        xg[...] = jnp.dot(d_mat[...], x_ref[...],
                          preferred_element_type=jnp.float32
                          ).astype(jnp.bfloat16)

        h = jnp.dot(xg[...], w1_ref[0].astype(jnp.bfloat16),
                    preferred_element_type=jnp.float32)
        h = jnp.maximum(h, 0.0).astype(jnp.bfloat16)
        contrib = jnp.dot(h, w2_ref[0].astype(jnp.bfloat16),
                          preferred_element_type=jnp.float32)

        w_row = w_t[pl.ds(e, 1)].astype(jnp.bfloat16)
        d_mat[...] = d_mat[...] * w_row
        s = lax.dot_general(d_mat[...], contrib.astype(jnp.bfloat16),
                            dimension_numbers=(((0,), (0,)), ((), ())),
                            preferred_element_type=jnp.float32)

        @pl.when(e == 0)
        def _():
            partial_ref[...] = s.astype(jnp.bfloat16)

        @pl.when(e != 0)
        def _():
            partial_ref[...] = (
                partial_ref[...].astype(jnp.float32) + s
            ).astype(jnp.bfloat16)

    grid_spec = pltpu.PrefetchScalarGridSpec(
        num_scalar_prefetch=1,
        grid=(E_LOC,),
        in_specs=[
            pl.BlockSpec((T, D), lambda e, q: (0, 0)),
            pl.BlockSpec((T, N_E), lambda e, q: (0, 0)),
            pl.BlockSpec((1, D, F_TILE), lambda e, q: (e, 0, q[0])),
            pl.BlockSpec((1, F_TILE, D), lambda e, q: (e, q[0], 0)),
        ],
        out_specs=pl.BlockSpec((T, D), lambda e, q: (0, 0)),
        scratch_shapes=[
            pltpu.VMEM((CAP, T), jnp.bfloat16),
            pltpu.VMEM((E_LOC, T), jnp.int32),
            pltpu.VMEM((E_LOC, T), jnp.float32),
            pltpu.VMEM((CAP, D), jnp.bfloat16),
        ],
    )
    return pl.pallas_call(
        body,
        grid_spec=grid_spec,
        out_shape=jax.ShapeDtypeStruct((T, D), jnp.bfloat16),
        compiler_params=pltpu.CompilerParams(
            dimension_semantics=("arbitrary",),
        ),
    )(q, x_all, gates, W1, W2)


def _combine_kernel(partial):

    def body(p_ref, out_ref, acc, sendb, comm_x, comm_y, comm_z,
             sx, rx, sy, ry, sz, rz):
        my_x = lax.axis_index("x")
        my_y = lax.axis_index("y")
        my_z = lax.axis_index("z")
        nbr_x = (1 - my_x, my_y, my_z)
        nbr_y = (my_x, 1 - my_y, my_z)
        nbr_z = (my_x, my_y, 1 - my_z)

        barrier_sem = pltpu.get_barrier_semaphore()
        for nbr in (nbr_x, nbr_y, nbr_z):
            pl.semaphore_signal(barrier_sem, inc=1, device_id=nbr,
                                device_id_type=pl.DeviceIdType.MESH)
        pl.semaphore_wait(barrier_sem, 3)

        rdma = pltpu.make_async_remote_copy(
            src_ref=p_ref.at[pl.ds((1 - my_x) * T_HALF, T_HALF)],
            dst_ref=comm_x, send_sem=sx, recv_sem=rx,
            device_id=nbr_x, device_id_type=pl.DeviceIdType.MESH,
        )
        rdma.start()
        rdma.wait()
        mine = p_ref[pl.ds(my_x * T_HALF, T_HALF), :]
        acc[...] = mine.astype(jnp.float32) + comm_x[...].astype(jnp.float32)

        sendb[...] = acc[...].astype(jnp.bfloat16)
        rdma = pltpu.make_async_remote_copy(
            src_ref=sendb, dst_ref=comm_y, send_sem=sy, recv_sem=ry,
            device_id=nbr_y, device_id_type=pl.DeviceIdType.MESH,
        )
        rdma.start()
        rdma.wait()
        acc[...] += comm_y[...].astype(jnp.float32)

        sendb[...] = acc[...].astype(jnp.bfloat16)
        rdma = pltpu.make_async_remote_copy(
            src_ref=sendb, dst_ref=comm_z, send_sem=sz, recv_sem=rz,
            device_id=nbr_z, device_id_type=pl.DeviceIdType.MESH,
        )
        rdma.start()
        rdma.wait()
        out_ref[...] = acc[...] + comm_z[...].astype(jnp.float32)

    return pl.pallas_call(
        body,
        out_shape=jax.ShapeDtypeStruct((T_HALF, D), jnp.float32),
        in_specs=[pl.BlockSpec(memory_space=pltpu.VMEM)],
        out_specs=pl.BlockSpec(memory_space=pltpu.VMEM),
        scratch_shapes=[
            pltpu.VMEM((T_HALF, D), jnp.float32),
            pltpu.VMEM((T_HALF, D), jnp.bfloat16),
            pltpu.VMEM((T_HALF, D), jnp.bfloat16),
            pltpu.VMEM((T_HALF, D), jnp.bfloat16),
            pltpu.VMEM((T_HALF, D), jnp.bfloat16),
            pltpu.SemaphoreType.DMA,
            pltpu.SemaphoreType.DMA,
            pltpu.SemaphoreType.DMA,
            pltpu.SemaphoreType.DMA,
            pltpu.SemaphoreType.DMA,
            pltpu.SemaphoreType.DMA,
        ],
        compiler_params=pltpu.CompilerParams(collective_id=1),
    )(partial)


def kernel(x, router, W1, W2):
    my_y = lax.axis_index("y")
    my_z = lax.axis_index("z")
    q = jnp.reshape(my_y * 2 + my_z, (1,)).astype(jnp.int32)

    x_all, gates = _exchange_kernel(x, router)
    partial = _moe_kernel(x_all, gates, W1, W2, q)
    return _combine_kernel(partial)


# device time: 151439 ns/iter; 1.5317x vs baseline; 1.1665x over previous
import jax
import jax.numpy as jnp
from jax import lax
from jax.experimental import pallas as pl
from jax.experimental.pallas import tpu as pltpu

T_HALF = 1024
T = 2 * T_HALF
D = 1024
F = 4096
F_TILE = 1024
N_F = F // F_TILE
E_LOC = 8
N_E = 2 * E_LOC
CAP = 320


def _exchange_kernel(x_shard, router_shard):

    def body(x_ref, r_ref, xo_ref, go_ref, xsend, rfull,
             sx, rx, sr, rr, sg, rg):
        my_x = lax.axis_index("x")
        my_y = lax.axis_index("y")
        my_z = lax.axis_index("z")
        nbr = (1 - my_x, my_y, my_z)

        barrier_sem = pltpu.get_barrier_semaphore()
        pl.semaphore_signal(barrier_sem, inc=1, device_id=nbr,
                            device_id_type=pl.DeviceIdType.MESH)
        pl.semaphore_wait(barrier_sem, 1)

        xsend[...] = x_ref[...].astype(jnp.bfloat16)
        rdma_x = pltpu.make_async_remote_copy(
            src_ref=xsend, dst_ref=xo_ref.at[pl.ds(my_x * T_HALF, T_HALF)],
            send_sem=sx, recv_sem=rx,
            device_id=nbr, device_id_type=pl.DeviceIdType.MESH,
        )
        rdma_x.start()
        xo_ref[pl.ds(my_x * T_HALF, T_HALF), :] = xsend[...]

        rfull[pl.ds(my_x, 1)] = r_ref[...][None]
        rdma_r = pltpu.make_async_remote_copy(
            src_ref=r_ref, dst_ref=rfull.at[my_x],
            send_sem=sr, recv_sem=rr,
            device_id=nbr, device_id_type=pl.DeviceIdType.MESH,
        )
        rdma_r.start()
        rdma_r.wait()

        router_full = jnp.concatenate([rfull[0], rfull[1]], axis=1)
        g_mine = jnp.dot(x_ref[...], router_full,
                         preferred_element_type=jnp.float32,
                         precision=lax.Precision.HIGHEST)
        go_ref[pl.ds(my_x * T_HALF, T_HALF), :] = g_mine
        rdma_g = pltpu.make_async_remote_copy(
            src_ref=go_ref.at[pl.ds(my_x * T_HALF, T_HALF)],
            dst_ref=go_ref.at[pl.ds(my_x * T_HALF, T_HALF)],
            send_sem=sg, recv_sem=rg,
            device_id=nbr, device_id_type=pl.DeviceIdType.MESH,
        )
        rdma_g.start()
        rdma_g.wait()
        rdma_x.wait()

    return pl.pallas_call(
        body,
        out_shape=[
            jax.ShapeDtypeStruct((T, D), jnp.bfloat16),
            jax.ShapeDtypeStruct((T, N_E), jnp.float32),
        ],
        in_specs=[
            pl.BlockSpec(memory_space=pltpu.VMEM),
            pl.BlockSpec(memory_space=pltpu.VMEM),
        ],
        out_specs=[
            pl.BlockSpec(memory_space=pltpu.VMEM),
            pl.BlockSpec(memory_space=pltpu.VMEM),
        ],
        scratch_shapes=[
            pltpu.VMEM((T_HALF, D), jnp.bfloat16),
            pltpu.VMEM((2, D, E_LOC), jnp.float32),
            pltpu.SemaphoreType.DMA,
            pltpu.SemaphoreType.DMA,
            pltpu.SemaphoreType.DMA,
            pltpu.SemaphoreType.DMA,
            pltpu.SemaphoreType.DMA,
            pltpu.SemaphoreType.DMA,
        ],
        compiler_params=pltpu.CompilerParams(collective_id=0),
    )(x_shard, router_shard)


def _moe_kernel(x_all, gates, W1, W2, q):

    def body(q_ref, x_ref, g_ref, w1_ref, w2_ref, partial_ref,
             d_mat, slot_t, w_t, xg):
        e = pl.program_id(0)
        my_x = lax.axis_index("x")

        @pl.when(e == 0)
        def _route_once():
            g_t = jnp.transpose(g_ref[...])
            ids = lax.broadcasted_iota(jnp.int32, (N_E, T), 0)
            m1 = jnp.max(g_t, axis=0, keepdims=True)
            top1 = jnp.argmax(g_t, axis=0).reshape(1, T)
            masked = jnp.where(ids == top1, -jnp.inf, g_t)
            m2 = jnp.max(masked, axis=0, keepdims=True)
            top2 = jnp.argmax(masked, axis=0).reshape(1, T)
            w1g = 1.0 / (1.0 + jnp.exp(m2 - m1))
            w2g = 1.0 - w1g

            eids = lax.broadcasted_iota(jnp.int32, (E_LOC, T), 0) \
                + my_x * E_LOC
            sel1 = top1 == eids
            sel2 = top2 == eids
            ind = sel1 | sel2
            w_t[...] = jnp.where(sel1, w1g, 0.0) + jnp.where(sel2, w2g, 0.0)

            pos = ind.astype(jnp.int32)
            k = 1
            while k < T:
                shifted = jnp.concatenate(
                    [jnp.zeros((E_LOC, k), jnp.int32), pos[:, :-k]], axis=1)
                pos = pos + shifted
                k *= 2
            slot_t[...] = jnp.where(ind, pos - 1, -1)

        cap_ids = lax.broadcasted_iota(jnp.int32, (CAP, T), 0)
        d_bool = cap_ids == slot_t[pl.ds(e, 1)]
        d_mat[...] = d_bool.astype(jnp.bfloat16)
        xg[...] = jnp.dot(d_mat[...], x_ref[...],
                          preferred_element_type=jnp.float32
                          ).astype(jnp.bfloat16)

        h = jnp.dot(xg[...], w1_ref[0].astype(jnp.bfloat16),
                    preferred_element_type=jnp.float32)
        h = jnp.maximum(h, 0.0).astype(jnp.bfloat16)
        contrib = jnp.dot(h, w2_ref[0].astype(jnp.bfloat16),
                          preferred_element_type=jnp.float32)

        w_row = w_t[pl.ds(e, 1)].astype(jnp.bfloat16)
        d_mat[...] = d_mat[...] * w_row
        s = lax.dot_general(d_mat[...], contrib.astype(jnp.bfloat16),
                            dimension_numbers=(((0,), (0,)), ((), ())),
                            preferred_element_type=jnp.float32)

        @pl.when(e == 0)
        def _():
            partial_ref[...] = s.astype(jnp.bfloat16)

        @pl.when(e != 0)
        def _():
            partial_ref[...] = (
                partial_ref[...].astype(jnp.float32) + s
            ).astype(jnp.bfloat16)

    grid_spec = pltpu.PrefetchScalarGridSpec(
        num_scalar_prefetch=1,
        grid=(E_LOC,),
        in_specs=[
            pl.BlockSpec((T, D), lambda e, q: (0, 0)),
            pl.BlockSpec((T, N_E), lambda e, q: (0, 0)),
            pl.BlockSpec((1, D, F_TILE), lambda e, q: (e, 0, q[0])),
            pl.BlockSpec((1, F_TILE, D), lambda e, q: (e, q[0], 0)),
        ],
        out_specs=pl.BlockSpec((T, D), lambda e, q: (0, 0)),
        scratch_shapes=[
            pltpu.VMEM((CAP, T), jnp.bfloat16),
            pltpu.VMEM((E_LOC, T), jnp.int32),
            pltpu.VMEM((E_LOC, T), jnp.float32),
            pltpu.VMEM((CAP, D), jnp.bfloat16),
        ],
    )
    return pl.pallas_call(
        body,
        grid_spec=grid_spec,
        out_shape=jax.ShapeDtypeStruct((T, D), jnp.bfloat16),
        compiler_params=pltpu.CompilerParams(
            dimension_semantics=("arbitrary",),
        ),
    )(q, x_all, gates, W1, W2)


def _combine_kernel(partial):

    n_ch = 4
    ch = T_HALF // n_ch

    def body(p_ref, out_ref, acc, sendb_y, sendb_z, comm_x, comm_y, comm_z,
             sx, rx, sy, ry, sz, rz):
        my_x = lax.axis_index("x")
        my_y = lax.axis_index("y")
        my_z = lax.axis_index("z")
        nbr_x = (1 - my_x, my_y, my_z)
        nbr_y = (my_x, 1 - my_y, my_z)
        nbr_z = (my_x, my_y, 1 - my_z)

        barrier_sem = pltpu.get_barrier_semaphore()
        for nbr in (nbr_x, nbr_y, nbr_z):
            pl.semaphore_signal(barrier_sem, inc=1, device_id=nbr,
                                device_id_type=pl.DeviceIdType.MESH)
        pl.semaphore_wait(barrier_sem, 3)

        def cs(c):
            return pl.ds(c * ch, ch)

        rd_x, rd_y, rd_z = [], [], []
        for c in range(n_ch):
            rd = pltpu.make_async_remote_copy(
                src_ref=p_ref.at[pl.ds((1 - my_x) * T_HALF + c * ch, ch)],
                dst_ref=comm_x.at[cs(c)], send_sem=sx.at[c], recv_sem=rx.at[c],
                device_id=nbr_x, device_id_type=pl.DeviceIdType.MESH,
            )
            rd.start()
            rd_x.append(rd)
        for c in range(n_ch):
            rd_x[c].wait()
            mine = p_ref[pl.ds(my_x * T_HALF + c * ch, ch), :]
            acc[cs(c), :] = (mine.astype(jnp.float32)
                             + comm_x[cs(c), :].astype(jnp.float32))
            sendb_y[cs(c), :] = acc[cs(c), :].astype(jnp.bfloat16)
            rd = pltpu.make_async_remote_copy(
                src_ref=sendb_y.at[cs(c)], dst_ref=comm_y.at[cs(c)],
                send_sem=sy.at[c], recv_sem=ry.at[c],
                device_id=nbr_y, device_id_type=pl.DeviceIdType.MESH,
            )
            rd.start()
            rd_y.append(rd)
        for c in range(n_ch):
            rd_y[c].wait()
            acc[cs(c), :] += comm_y[cs(c), :].astype(jnp.float32)
            sendb_z[cs(c), :] = acc[cs(c), :].astype(jnp.bfloat16)
            rd = pltpu.make_async_remote_copy(
                src_ref=sendb_z.at[cs(c)], dst_ref=comm_z.at[cs(c)],
                send_sem=sz.at[c], recv_sem=rz.at[c],
                device_id=nbr_z, device_id_type=pl.DeviceIdType.MESH,
            )
            rd.start()
            rd_z.append(rd)
        for c in range(n_ch):
            rd_z[c].wait()
            out_ref[cs(c), :] = acc[cs(c), :] + comm_z[cs(c), :].astype(
                jnp.float32)

    return pl.pallas_call(
        body,
        out_shape=jax.ShapeDtypeStruct((T_HALF, D), jnp.float32),
        in_specs=[pl.BlockSpec(memory_space=pltpu.VMEM)],
        out_specs=pl.BlockSpec(memory_space=pltpu.VMEM),
        scratch_shapes=[
            pltpu.VMEM((T_HALF, D), jnp.float32),
            pltpu.VMEM((T_HALF, D), jnp.bfloat16),
            pltpu.VMEM((T_HALF, D), jnp.bfloat16),
            pltpu.VMEM((T_HALF, D), jnp.bfloat16),
            pltpu.VMEM((T_HALF, D), jnp.bfloat16),
            pltpu.VMEM((T_HALF, D), jnp.bfloat16),
            pltpu.SemaphoreType.DMA((n_ch,)),
            pltpu.SemaphoreType.DMA((n_ch,)),
            pltpu.SemaphoreType.DMA((n_ch,)),
            pltpu.SemaphoreType.DMA((n_ch,)),
            pltpu.SemaphoreType.DMA((n_ch,)),
            pltpu.SemaphoreType.DMA((n_ch,)),
        ],
        compiler_params=pltpu.CompilerParams(collective_id=1),
    )(partial)


def kernel(x, router, W1, W2):
    my_y = lax.axis_index("y")
    my_z = lax.axis_index("z")
    q = jnp.reshape(my_y * 2 + my_z, (1,)).astype(jnp.int32)

    x_all, gates = _exchange_kernel(x, router)
    partial = _moe_kernel(x_all, gates, W1, W2, q)
    return _combine_kernel(partial)
